# baseline (speedup 1.0000x reference)
import jax
import jax.numpy as jnp
from jax.experimental import pallas as pl

_N_EXPERTS = 64
_K = 8


def _router_block(x1_ref, x2_ref, w_ref, b_ref, probs_ref, mask_ref):
    w = w_ref[...]
    b_col = b_ref[:, 0:1]
    d_half = x1_ref.shape[1]
    w1 = w[:, :d_half]
    w2 = w[:, d_half:]
    dn = (((1,), (1,)), ((), ()))
    logits_t = (
        jax.lax.dot_general(w1, x1_ref[...], dimension_numbers=dn,
                            preferred_element_type=jnp.float32)
        + jax.lax.dot_general(w2, x2_ref[...], dimension_numbers=dn,
                              preferred_element_type=jnp.float32)
        + b_col
    )

    m = jnp.max(logits_t, axis=0, keepdims=True)
    e = jnp.exp(logits_t - m)
    s = jnp.sum(e, axis=0, keepdims=True)
    probs_t = e / s

    iota = jax.lax.broadcasted_iota(jnp.int32, logits_t.shape, 0)
    work = logits_t
    mask_t = jnp.zeros(logits_t.shape, dtype=jnp.float32)
    for _ in range(_K):
        mx = jnp.max(work, axis=0, keepdims=True)
        cand = jnp.where(work == mx, iota, _N_EXPERTS)
        first = jnp.min(cand, axis=0, keepdims=True)
        sel = iota == first
        mask_t = jnp.where(sel, 1.0, mask_t)
        work = jnp.where(sel, -jnp.inf, work)

    probs_ref[...] = probs_t.T
    mask_ref[...] = mask_t.T > 0.5


@jax.jit
def kernel(x, W, b):
    n_tokens, d_model = x.shape
    block_t = 4096
    d_half = d_model // 2
    grid = (n_tokens // block_t,)
    b2d = jnp.broadcast_to(b[:, None], (_N_EXPERTS, 128))

    probs, mask = pl.pallas_call(
        _router_block,
        grid=grid,
        in_specs=[
            pl.BlockSpec((block_t, d_half), lambda i: (i, 0)),
            pl.BlockSpec((block_t, d_half), lambda i: (i, 1)),
            pl.BlockSpec((_N_EXPERTS, d_model), lambda i: (0, 0)),
            pl.BlockSpec((_N_EXPERTS, 128), lambda i: (0, 0)),
        ],
        out_specs=[
            pl.BlockSpec((block_t, _N_EXPERTS), lambda i: (i, 0)),
            pl.BlockSpec((block_t, _N_EXPERTS), lambda i: (i, 0)),
        ],
        out_shape=[
            jax.ShapeDtypeStruct((n_tokens, _N_EXPERTS), jnp.float32),
            jax.ShapeDtypeStruct((n_tokens, _N_EXPERTS), jnp.bool_),
        ],
    )(x, x, W, b2d)
    return (probs, mask)
